# Initial kernel scaffold; baseline (speedup 1.0000x reference)
import jax
import jax.numpy as jnp
from jax.experimental import pallas as pl


def kernel(predict, y):
    raise NotImplementedError("write your pallas kernel here")


# speedup vs baseline: 13.3141x; 13.3141x over previous
import functools

import jax
import jax.numpy as jnp
from jax import lax
from jax.experimental import pallas as pl

_N = 65536
_ROWS = 512
_COLS = 128


def _ohem_kernel(x_ref, y_ref, out_ref):
    x = x_ref[...]
    yv = y_ref[...]

    loss = jnp.maximum(x, 0.0) - x * yv + jnp.log1p(jnp.exp(-jnp.abs(x)))

    m1 = jax.nn.sigmoid(x) >= 0.5
    ones = jnp.ones_like(x)
    n1 = jnp.sum(jnp.where(m1, ones, 0.0)).astype(jnp.int32)
    n0 = jnp.int32(_N) - n1
    sum1 = jnp.sum(jnp.where(m1, loss, 0.0))
    sum0 = jnp.sum(jnp.where(m1, 0.0, loss))

    min_n = jnp.minimum(n0, n1)
    max_cap = jnp.maximum(min_n, 3 * (min_n + 1))

    big_is_1 = n1 > n0
    c_big = jnp.maximum(n0, n1)
    k_big = jnp.minimum(max_cap, c_big)

    bits = lax.bitcast_convert_type(loss, jnp.int32)
    neg1 = jnp.full_like(bits, -1)
    big_bits = jnp.where(big_is_1, jnp.where(m1, bits, neg1),
                         jnp.where(m1, neg1, bits))

    def step(j, t):
        cand = t | (jnp.int32(1) << (jnp.int32(30) - j))
        cnt = jnp.sum(jnp.where(big_bits >= cand, ones, 0.0)).astype(jnp.int32)
        return jnp.where(cnt >= k_big, cand, t)

    t_bits = lax.fori_loop(0, 31, step, jnp.int32(0))
    t_val = lax.bitcast_convert_type(t_bits, jnp.float32)

    gt = big_bits > t_bits
    cnt_gt = jnp.sum(jnp.where(gt, ones, 0.0)).astype(jnp.int32)
    sum_gt = jnp.sum(jnp.where(gt, loss, 0.0))
    topk_sum = sum_gt + (k_big - cnt_gt).astype(jnp.float32) * t_val
    mean_top = topk_sum / jnp.maximum(k_big, 1).astype(jnp.float32)

    mean_all0 = sum0 / jnp.maximum(n0, 1).astype(jnp.float32)
    mean_all1 = sum1 / jnp.maximum(n1, 1).astype(jnp.float32)

    zero = jnp.float32(0.0)
    term0 = jnp.where(n0 == min_n,
                      jnp.where(n0 > 0, mean_all0, zero), mean_top)
    term1 = jnp.where(n1 == min_n,
                      jnp.where(n1 > 0, mean_all1, zero), mean_top)
    axis = (n0 > 0).astype(jnp.float32) + (n1 > 0).astype(jnp.float32)
    out_ref[...] = ((term0 + term1) / axis).reshape(1, 1)


@functools.partial(jax.jit, static_argnames=("interpret",))
def kernel(predict, y, interpret=False):
    x = predict.reshape(_ROWS, _COLS)
    yv = y.reshape(_ROWS, _COLS)
    out = pl.pallas_call(
        _ohem_kernel,
        out_shape=jax.ShapeDtypeStruct((1, 1), jnp.float32),
        interpret=interpret,
    )(x, yv)
    return out[0, 0]
